# baseline (speedup 1.0000x reference)
import jax
import jax.numpy as jnp
from jax.experimental import pallas as pl
from jax.experimental.pallas import tpu as pltpu

V, H, E = 32, 1024, 1536
N = 64


def _copy_body(sids_smem, order_smem, in_l, in_r, out_ref):
    out_ref[:, :, : E // 2] = in_l[...]
    out_ref[:, :, E // 2 :] = in_r[...]


def _tc_gather(sids, order, table):
    return pl.pallas_call(
        _copy_body,
        grid_spec=pltpu.PrefetchScalarGridSpec(
            num_scalar_prefetch=2,
            grid=(N,),
            in_specs=[
                pl.BlockSpec((1, H, E // 2), lambda i, sids, order: (sids[i], 0, 0)),
                pl.BlockSpec((1, H, E // 2), lambda i, sids, order: (sids[i], 0, 1)),
            ],
            out_specs=pl.BlockSpec((1, H, E), lambda i, sids, order: (order[i], 0, 0)),
        ),
        out_shape=jax.ShapeDtypeStruct((N, H, E), jnp.float32),
    )(sids, order, table, table)


def kernel(cat_ids, W):
    ids = cat_ids.astype(jnp.int32)
    order = jnp.argsort(ids).astype(jnp.int32)
    sids = ids[order]
    return _tc_gather(sids, order, W)


# speedup vs baseline: 1.0042x; 1.0042x over previous
import jax
import jax.numpy as jnp
from jax.experimental import pallas as pl
from jax.experimental.pallas import tpu as pltpu

V, H, E = 32, 1024, 1536
N = 64


def _copy_body(sids_smem, order_smem, in_ref, out_ref):
    out_ref[...] = in_ref[...]


def _tc_gather(sids, order, table):
    return pl.pallas_call(
        _copy_body,
        grid_spec=pltpu.PrefetchScalarGridSpec(
            num_scalar_prefetch=2,
            grid=(N,),
            in_specs=[
                pl.BlockSpec((1, H, E), lambda i, sids, order: (sids[i], 0, 0)),
            ],
            out_specs=pl.BlockSpec((1, H, E), lambda i, sids, order: (order[i], 0, 0)),
        ),
        out_shape=jax.ShapeDtypeStruct((N, H, E), jnp.float32),
    )(sids, order, table)


def kernel(cat_ids, W):
    ids = cat_ids.astype(jnp.int32)
    order = jnp.argsort(ids).astype(jnp.int32)
    sids = ids[order]
    return _tc_gather(sids, order, W)
